# baseline (speedup 1.0000x reference)
import functools

import jax
import jax.numpy as jnp
from jax import lax
from jax.experimental import pallas as pl
from jax.experimental.pallas import tpu as pltpu
from jax.experimental.pallas import tpu_sc as plsc

B = 16384
EMB = 64
OUT = 512
N_SINGERS = 417
N_GENRES = 18
NC = 2
NS = 16
NW = NC * NS
BPW = B // NW
CHUNK = 128
NCHUNK = BPW // CHUNK


NSPLIT = 2
BS = B // NSPLIT
SPW = BS // NW
SNCHUNK = SPW // CHUNK


def _sc_gather_body(em_hbm, idm_hbm, om_hbm, idx_v, rows_v, sem):
    wid = lax.axis_index("s") * NC + lax.axis_index("c")
    row0 = wid * SNCHUNK

    pltpu.sync_copy(idm_hbm.at[pl.ds(row0, SNCHUNK)], idx_v)
    copies = []
    for j in range(SNCHUNK):
        dst = rows_v.at[pl.ds(j * CHUNK, CHUNK)]
        copies.append(pltpu.async_copy(em_hbm.at[idx_v.at[j]], dst, sem))
    for c in copies:
        c.wait()
    pltpu.sync_copy(rows_v, om_hbm.at[pl.ds(wid * SPW, SPW)])


@jax.jit
def _sc_gather(E_music, idm):
    mesh = plsc.VectorSubcoreMesh(core_axis_name="c", subcore_axis_name="s",
                                  num_cores=NC, num_subcores=NS)
    k = pl.kernel(_sc_gather_body,
                  out_type=jax.ShapeDtypeStruct((BS, EMB), jnp.float32),
                  mesh=mesh,
                  scratch_types=[
                      pltpu.VMEM((SNCHUNK, CHUNK), jnp.int32),
                      pltpu.VMEM((SPW, EMB), jnp.float32),
                      pltpu.SemaphoreType.DMA,
                  ],
                  compiler_params=pltpu.CompilerParams(
                      use_tc_tiling_on_sc=False))
    return k(E_music, idm)


def _mm_body(m_ref, sidx_ref, gidx_ref, es_ref, eg_ref,
             w1_ref, w2_ref, w3_ref, b_ref, o_ref):
    bb = m_ref.shape[0]
    sidx = sidx_ref[0, 0, :]
    gidx = gidx_ref[0, 0, :]
    s_oh = (sidx[:, None] ==
            lax.broadcasted_iota(jnp.int32, (bb, N_SINGERS), 1)
            ).astype(jnp.float32)
    g_oh = (gidx[:, None] ==
            lax.broadcasted_iota(jnp.int32, (bb, N_GENRES), 1)
            ).astype(jnp.float32)
    s_emb = jnp.dot(s_oh, es_ref[...], preferred_element_type=jnp.float32)
    g_emb = jnp.dot(g_oh, eg_ref[...], preferred_element_type=jnp.float32)
    acc = jnp.dot(m_ref[...], w1_ref[...], preferred_element_type=jnp.float32)
    acc += jnp.dot(s_emb, w2_ref[...], preferred_element_type=jnp.float32)
    acc += jnp.dot(g_emb, w3_ref[...], preferred_element_type=jnp.float32)
    o_ref[...] = acc + b_ref[...]


@functools.partial(jax.jit, static_argnames=("bb", "k", "aliased"))
def _tc_project(memb, sidx, gidx, E_singer, E_genre, w1, w2, w3, b, prev,
                bb=1024, k=0, aliased=False):
    hb = BS // bb
    grid = (hb,)
    idx_spec = pl.BlockSpec((1, 1, bb), lambda i: (i, 0, 0))
    w_spec = pl.BlockSpec((EMB, OUT), lambda i: (0, 0))
    body = _mm_body if not aliased else (
        lambda m, s, g, es, eg, a, b_, c, d, _p, o:
        _mm_body(m, s, g, es, eg, a, b_, c, d, o))
    in_specs = [
        pl.BlockSpec((bb, EMB), lambda i: (i, 0)),
        idx_spec, idx_spec,
        pl.BlockSpec((N_SINGERS, EMB), lambda i: (0, 0)),
        pl.BlockSpec((N_GENRES, EMB), lambda i: (0, 0)),
        w_spec, w_spec, w_spec,
        pl.BlockSpec((1, OUT), lambda i: (0, 0)),
    ]
    args = [memb, sidx, gidx, E_singer, E_genre, w1, w2, w3, b]
    kwargs = {}
    if aliased:
        in_specs.append(pl.BlockSpec(memory_space=pl.ANY))
        args.append(prev)
        kwargs["input_output_aliases"] = {9: 0}
    return pl.pallas_call(
        body,
        grid=grid,
        in_specs=in_specs,
        out_specs=pl.BlockSpec((bb, OUT), lambda i: (i + k * hb, 0)),
        out_shape=jax.ShapeDtypeStruct((B, OUT), jnp.float32),
        **kwargs,
    )(*args)


def kernel(lyric, features, singer, genre, id, W_feat, b_feat,
           E_singer, E_genre, E_music, W_out, b_out):
    bb = 1024
    idm = id.astype(jnp.int32).reshape(NSPLIT, BS // CHUNK, CHUNK)
    sidx = singer.astype(jnp.int32).reshape(NSPLIT, BS // bb, 1, bb)
    gidx = genre.astype(jnp.int32).reshape(NSPLIT, BS // bb, 1, bb)
    WT = W_out.T
    w1, w2, w3 = WT[:EMB], WT[EMB:2 * EMB], WT[2 * EMB:]
    b2 = b_out.reshape(1, OUT)
    membs = [_sc_gather(E_music, idm[k]) for k in range(NSPLIT)]
    out = None
    for k in range(NSPLIT):
        out = _tc_project(membs[k], sidx[k], gidx[k], E_singer, E_genre,
                          w1, w2, w3, b2, out, bb=bb, k=k, aliased=k > 0)
    return out
